# baseline (speedup 1.0000x reference)
import functools

import jax
import jax.numpy as jnp
from jax import lax
from jax.experimental import pallas as pl
from jax.experimental.pallas import tpu as pltpu
from jax.experimental.pallas import tpu_sc as plsc

U_DIM = 64
NT = 3
VBLK = 65536
RBLK = VBLK // 128


def _tc_project(tblT, wT):
    D, V = tblT.shape
    nblk = pl.cdiv(V, VBLK)

    def body(t_ref, w_ref, o0_ref, o1_ref, o2_ref):
        p = lax.dot_general(w_ref[...], t_ref[...], (((1,), (0,)), ((), ())),
                            preferred_element_type=jnp.float32)
        o0_ref[...] = p[0:1, :].reshape(RBLK, 128)
        o1_ref[...] = p[1:2, :].reshape(RBLK, 128)
        o2_ref[...] = p[2:3, :].reshape(RBLK, 128)

    osd = jax.ShapeDtypeStruct((nblk * RBLK, 128), jnp.float32)
    ospec = pl.BlockSpec((RBLK, 128), lambda g: (g, 0))
    return pl.pallas_call(
        body,
        grid=(nblk,),
        in_specs=[
            pl.BlockSpec((D, VBLK), lambda g: (0, g)),
            pl.BlockSpec((NT, D), lambda g: (0, 0)),
        ],
        out_specs=(ospec, ospec, ospec),
        out_shape=(osd, osd, osd),
    )(tblT, wT)


def _sc_gather(pu, uid, pi, iid):
    B = uid.shape[0]
    info = plsc.get_sparse_core_info()
    NC, NS = info.num_cores, info.num_subcores
    NW = NC * NS
    bpw = B // NW
    nchunk = bpw // 16
    mesh = plsc.VectorSubcoreMesh(core_axis_name="c", subcore_axis_name="s")
    osd = jax.ShapeDtypeStruct((B,), jnp.float32)

    @functools.partial(
        pl.kernel,
        mesh=mesh,
        compiler_params=pltpu.CompilerParams(use_tc_tiling_on_sc=True,
                                             needs_layout_passes=False),
        out_type=(osd,) * (2 * NT),
        scratch_types=[
            pltpu.VMEM((bpw,), jnp.int32),
            pltpu.VMEM((bpw,), jnp.int32),
            pltpu.VMEM((bpw,), jnp.int32),
            pltpu.VMEM((bpw,), jnp.float32),
            pltpu.VMEM((bpw, 128), jnp.float32),
            pltpu.SemaphoreType.DMA,
        ],
    )
    def gather_k(pu0, pu1, pu2, uid_hbm, pi0, pi1, pi2, iid_hbm,
                 eu0, eu1, eu2, ei0, ei1, ei2,
                 ids_v, row_v, lane_v, val_v, rows_v, sem):
        wid = lax.axis_index("s") * NC + lax.axis_index("c")
        base = wid * bpw
        for id_hbm, srcs, dsts in (
            (uid_hbm, (pu0, pu1, pu2), (eu0, eu1, eu2)),
            (iid_hbm, (pi0, pi1, pi2), (ei0, ei1, ei2)),
        ):
            pltpu.sync_copy(id_hbm.at[pl.ds(base, bpw)], ids_v)
            for j in range(nchunk):
                ids16 = ids_v[pl.ds(16 * j, 16)]
                row_v[pl.ds(16 * j, 16)] = ids16 >> 7
                lane_v[pl.ds(16 * j, 16)] = ids16 & 127
            for src, dst in zip(srcs, dsts):
                pltpu.async_copy(src.at[row_v], rows_v, sem).wait()
                for j in range(nchunk):
                    r16 = lax.iota(jnp.int32, 16) + 16 * j
                    vals = plsc.load_gather(
                        rows_v, [r16, lane_v[pl.ds(16 * j, 16)]])
                    val_v[pl.ds(16 * j, 16)] = vals
                pltpu.sync_copy(val_v, dst.at[pl.ds(base, bpw)])

    return gather_k(pu[0], pu[1], pu[2], uid, pi[0], pi[1], pi[2], iid)


def _tc_combine(eu, ei, uf, W_ufT, b_uf, W_finalT, b_finalT):
    B, ufd = uf.shape
    blk = 2048
    espec = pl.BlockSpec((blk // 128, 128), lambda i: (i, 0))

    def body(eu0_ref, eu1_ref, eu2_ref, ei0_ref, ei1_ref, ei2_ref,
             uf_ref, wufT_ref, buf_ref, wfT_ref, bfT_ref, out_ref):
        rows = [
            (u_ref[...] + i_ref[...]).reshape(1, blk)
            for u_ref, i_ref in ((eu0_ref, ei0_ref), (eu1_ref, ei1_ref),
                                 (eu2_ref, ei2_ref))
        ]
        emb = jnp.concatenate(rows, axis=0)
        tT = lax.dot_general(wufT_ref[...], uf_ref[...],
                             (((1,), (1,)), ((), ())),
                             preferred_element_type=jnp.float32) + buf_ref[...]
        emb += jnp.dot(wfT_ref[...][:, U_DIM:2 * U_DIM], tT,
                       preferred_element_type=jnp.float32)
        out_ref[...] = emb + bfT_ref[...]

    outT = pl.pallas_call(
        body,
        grid=(B // blk,),
        in_specs=[
            espec, espec, espec, espec, espec, espec,
            pl.BlockSpec((blk, ufd), lambda i: (i, 0)),
            pl.BlockSpec((U_DIM, ufd), lambda i: (0, 0)),
            pl.BlockSpec((U_DIM, 1), lambda i: (0, 0)),
            pl.BlockSpec((NT, 3 * U_DIM), lambda i: (0, 0)),
            pl.BlockSpec((NT, 1), lambda i: (0, 0)),
        ],
        out_specs=pl.BlockSpec((NT, blk), lambda i: (0, i)),
        out_shape=jax.ShapeDtypeStruct((NT, B), jnp.float32),
    )(eu[0].reshape(128, -1), eu[1].reshape(128, -1), eu[2].reshape(128, -1),
      ei[0].reshape(128, -1), ei[1].reshape(128, -1), ei[2].reshape(128, -1),
      uf, W_ufT, b_uf, W_finalT, b_finalT)
    return outT.T


def kernel(user_id, user_features, item_id, user_table, item_table,
           W_uf, b_uf, W_final, b_final):
    uid = user_id.astype(jnp.int32)
    iid = item_id.astype(jnp.int32)
    pu = _tc_project(user_table.T, W_final[0:U_DIM, :].T)
    pi = _tc_project(item_table.T, W_final[2 * U_DIM:, :].T)
    g = _sc_gather(pu, uid, pi, iid)
    return _tc_combine(g[:3], g[3:], user_features, W_uf.T,
                       b_uf.reshape(-1, 1), W_final.T,
                       b_final.reshape(-1, 1))


# speedup vs baseline: 1.0151x; 1.0151x over previous
import functools

import jax
import jax.numpy as jnp
from jax import lax
from jax.experimental import pallas as pl
from jax.experimental.pallas import tpu as pltpu
from jax.experimental.pallas import tpu_sc as plsc

U_DIM = 64
NT = 3
VBLK = 32768
RBLK = VBLK // 128


def _tc_project(tblT, wT):
    D, V = tblT.shape
    nblk = pl.cdiv(V, VBLK)

    def body(t_ref, w_ref, o0_ref, o1_ref, o2_ref):
        p = lax.dot_general(w_ref[...], t_ref[...], (((1,), (0,)), ((), ())),
                            preferred_element_type=jnp.float32)
        o0_ref[...] = p[0:1, :].reshape(RBLK, 128)
        o1_ref[...] = p[1:2, :].reshape(RBLK, 128)
        o2_ref[...] = p[2:3, :].reshape(RBLK, 128)

    osd = jax.ShapeDtypeStruct((nblk * RBLK, 128), jnp.float32)
    ospec = pl.BlockSpec((RBLK, 128), lambda g: (g, 0))
    return pl.pallas_call(
        body,
        grid=(nblk,),
        in_specs=[
            pl.BlockSpec((D, VBLK), lambda g: (0, g)),
            pl.BlockSpec((NT, D), lambda g: (0, 0)),
        ],
        out_specs=(ospec, ospec, ospec),
        out_shape=(osd, osd, osd),
    )(tblT, wT)


def _sc_gather(p3, ids):
    B = ids.shape[0]
    info = plsc.get_sparse_core_info()
    NC, NS = info.num_cores, info.num_subcores
    NW = NC * NS
    bpw = B // NW
    nchunk = bpw // 16
    mesh = plsc.VectorSubcoreMesh(core_axis_name="c", subcore_axis_name="s")
    osd = jax.ShapeDtypeStruct((B,), jnp.float32)

    @functools.partial(
        pl.kernel,
        mesh=mesh,
        compiler_params=pltpu.CompilerParams(use_tc_tiling_on_sc=True,
                                             needs_layout_passes=False),
        out_type=(osd,) * NT,
        scratch_types=[
            pltpu.VMEM((bpw,), jnp.int32),
            pltpu.VMEM((bpw,), jnp.int32),
            pltpu.VMEM((bpw,), jnp.int32),
            pltpu.VMEM((bpw,), jnp.float32),
            pltpu.VMEM((bpw, 128), jnp.float32),
            pltpu.SemaphoreType.DMA,
        ],
    )
    def gather_k(p0, p1, p2, id_hbm, e0, e1, e2,
                 ids_v, row_v, lane_v, val_v, rows_v, sem):
        wid = lax.axis_index("s") * NC + lax.axis_index("c")
        base = wid * bpw
        pltpu.sync_copy(id_hbm.at[pl.ds(base, bpw)], ids_v)
        for j in range(nchunk):
            ids16 = ids_v[pl.ds(16 * j, 16)]
            row_v[pl.ds(16 * j, 16)] = ids16 >> 7
            lane_v[pl.ds(16 * j, 16)] = ids16 & 127
        for src, dst in ((p0, e0), (p1, e1), (p2, e2)):
            pltpu.async_copy(src.at[row_v], rows_v, sem).wait()
            for j in range(nchunk):
                r16 = lax.iota(jnp.int32, 16) + 16 * j
                vals = plsc.load_gather(
                    rows_v, [r16, lane_v[pl.ds(16 * j, 16)]])
                val_v[pl.ds(16 * j, 16)] = vals
            pltpu.sync_copy(val_v, dst.at[pl.ds(base, bpw)])

    return gather_k(p3[0], p3[1], p3[2], ids)


def _tc_combine(eu, ei, uf, W_ufT, b_uf, W_finalT, b_finalT):
    B, ufd = uf.shape
    blk = 2048
    espec = pl.BlockSpec((blk // 128, 128), lambda i: (i, 0))

    def body(eu0_ref, eu1_ref, eu2_ref, ei0_ref, ei1_ref, ei2_ref,
             uf_ref, wufT_ref, buf_ref, wfT_ref, bfT_ref, out_ref):
        rows = [
            (u_ref[...] + i_ref[...]).reshape(1, blk)
            for u_ref, i_ref in ((eu0_ref, ei0_ref), (eu1_ref, ei1_ref),
                                 (eu2_ref, ei2_ref))
        ]
        emb = jnp.concatenate(rows, axis=0)
        tT = lax.dot_general(wufT_ref[...], uf_ref[...],
                             (((1,), (1,)), ((), ())),
                             preferred_element_type=jnp.float32) + buf_ref[...]
        emb += jnp.dot(wfT_ref[...][:, U_DIM:2 * U_DIM], tT,
                       preferred_element_type=jnp.float32)
        out_ref[...] = emb + bfT_ref[...]

    outT = pl.pallas_call(
        body,
        grid=(B // blk,),
        in_specs=[
            espec, espec, espec, espec, espec, espec,
            pl.BlockSpec((blk, ufd), lambda i: (i, 0)),
            pl.BlockSpec((U_DIM, ufd), lambda i: (0, 0)),
            pl.BlockSpec((U_DIM, 1), lambda i: (0, 0)),
            pl.BlockSpec((NT, 3 * U_DIM), lambda i: (0, 0)),
            pl.BlockSpec((NT, 1), lambda i: (0, 0)),
        ],
        out_specs=pl.BlockSpec((NT, blk), lambda i: (0, i)),
        out_shape=jax.ShapeDtypeStruct((NT, B), jnp.float32),
    )(eu[0].reshape(128, -1), eu[1].reshape(128, -1), eu[2].reshape(128, -1),
      ei[0].reshape(128, -1), ei[1].reshape(128, -1), ei[2].reshape(128, -1),
      uf, W_ufT, b_uf, W_finalT, b_finalT)
    return outT.T


def kernel(user_id, user_features, item_id, user_table, item_table,
           W_uf, b_uf, W_final, b_final):
    uid = user_id.astype(jnp.int32)
    iid = item_id.astype(jnp.int32)
    pi = _tc_project(item_table.T, W_final[2 * U_DIM:, :].T)
    ei = _sc_gather(pi, iid)
    pu = _tc_project(user_table.T, W_final[0:U_DIM, :].T)
    eu = _sc_gather(pu, uid)
    return _tc_combine(eu, ei, user_features, W_uf.T,
                       b_uf.reshape(-1, 1), W_final.T,
                       b_final.reshape(-1, 1))


# speedup vs baseline: 1.0231x; 1.0079x over previous
import functools

import jax
import jax.numpy as jnp
from jax import lax
from jax.experimental import pallas as pl
from jax.experimental.pallas import tpu as pltpu
from jax.experimental.pallas import tpu_sc as plsc

U_DIM = 64
NT = 3
VBLK = 32768
RBLK = VBLK // 128


def _tc_project(tblT, wT):
    D, V = tblT.shape
    nblk = pl.cdiv(V, VBLK)

    def body(t_ref, w_ref, o0_ref, o1_ref, o2_ref):
        p = lax.dot_general(w_ref[...], t_ref[...], (((1,), (0,)), ((), ())),
                            preferred_element_type=jnp.float32)
        o0_ref[...] = p[0:1, :].reshape(RBLK, 128)
        o1_ref[...] = p[1:2, :].reshape(RBLK, 128)
        o2_ref[...] = p[2:3, :].reshape(RBLK, 128)

    osd = jax.ShapeDtypeStruct((nblk * RBLK, 128), jnp.float32)
    ospec = pl.BlockSpec((RBLK, 128), lambda g: (g, 0))
    return pl.pallas_call(
        body,
        grid=(nblk,),
        in_specs=[
            pl.BlockSpec((D, VBLK), lambda g: (0, g)),
            pl.BlockSpec((NT, D), lambda g: (0, 0)),
        ],
        out_specs=(ospec, ospec, ospec),
        out_shape=(osd, osd, osd),
    )(tblT, wT)


def _sc_gather(p3, ids):
    B = ids.shape[0]
    info = plsc.get_sparse_core_info()
    NC, NS = info.num_cores, info.num_subcores
    NW = NC * NS
    bpw = B // NW
    nchunk = bpw // 16
    mesh = plsc.VectorSubcoreMesh(core_axis_name="c", subcore_axis_name="s")
    osd = jax.ShapeDtypeStruct((B,), jnp.float32)

    nq = 4
    qs = bpw // nq
    qchunk = qs // 16

    @functools.partial(
        pl.kernel,
        mesh=mesh,
        compiler_params=pltpu.CompilerParams(use_tc_tiling_on_sc=True,
                                             needs_layout_passes=False),
        out_type=(osd,) * NT,
        scratch_types=[
            pltpu.VMEM((bpw,), jnp.int32),
            pltpu.VMEM((bpw,), jnp.int32),
            pltpu.VMEM((bpw,), jnp.int32),
            pltpu.VMEM((qs,), jnp.float32),
            pltpu.VMEM((qs, 128), jnp.float32),
            pltpu.VMEM((qs, 128), jnp.float32),
            pltpu.SemaphoreType.DMA,
            pltpu.SemaphoreType.DMA,
        ],
    )
    def gather_k(p0, p1, p2, id_hbm, e0, e1, e2,
                 ids_v, row_v, lane_v, val_v, rows_a, rows_b, sem_a, sem_b):
        wid = lax.axis_index("s") * NC + lax.axis_index("c")
        base = wid * bpw
        pltpu.sync_copy(id_hbm.at[pl.ds(base, bpw)], ids_v)
        for j in range(nchunk):
            ids16 = ids_v[pl.ds(16 * j, 16)]
            row_v[pl.ds(16 * j, 16)] = ids16 >> 7
            lane_v[pl.ds(16 * j, 16)] = ids16 & 127
        tasks = ((p0, e0), (p1, e1), (p2, e2))
        seq = [(t, q) for t in range(NT) for q in range(nq)]
        bufs = ((rows_a, sem_a), (rows_b, sem_b))

        def fire(k):
            t, q = seq[k]
            buf, sem = bufs[k % 2]
            return pltpu.async_copy(
                tasks[t][0].at[row_v.at[pl.ds(q * qs, qs)]], buf, sem)

        pending = [fire(0), fire(1)]
        for k, (t, q) in enumerate(seq):
            pending[k % 2].wait()
            buf, _ = bufs[k % 2]
            for j in range(qchunk):
                r16 = lax.iota(jnp.int32, 16) + 16 * j
                vals = plsc.load_gather(
                    buf, [r16, lane_v[pl.ds(q * qs + 16 * j, 16)]])
                val_v[pl.ds(16 * j, 16)] = vals
            pltpu.sync_copy(val_v, tasks[t][1].at[pl.ds(base + q * qs, qs)])
            if k + 2 < len(seq):
                pending[k % 2] = fire(k + 2)

    return gather_k(p3[0], p3[1], p3[2], ids)


def _tc_combine(eu, ei, uf, W_ufT, b_uf, W_finalT, b_finalT):
    B, ufd = uf.shape
    blk = 2048
    espec = pl.BlockSpec((blk // 128, 128), lambda i: (i, 0))

    def body(eu0_ref, eu1_ref, eu2_ref, ei0_ref, ei1_ref, ei2_ref,
             uf_ref, wufT_ref, buf_ref, wfT_ref, bfT_ref, out_ref):
        rows = [
            (u_ref[...] + i_ref[...]).reshape(1, blk)
            for u_ref, i_ref in ((eu0_ref, ei0_ref), (eu1_ref, ei1_ref),
                                 (eu2_ref, ei2_ref))
        ]
        emb = jnp.concatenate(rows, axis=0)
        tT = lax.dot_general(wufT_ref[...], uf_ref[...],
                             (((1,), (1,)), ((), ())),
                             preferred_element_type=jnp.float32) + buf_ref[...]
        emb += jnp.dot(wfT_ref[...][:, U_DIM:2 * U_DIM], tT,
                       preferred_element_type=jnp.float32)
        out_ref[...] = emb + bfT_ref[...]

    outT = pl.pallas_call(
        body,
        grid=(B // blk,),
        in_specs=[
            espec, espec, espec, espec, espec, espec,
            pl.BlockSpec((blk, ufd), lambda i: (i, 0)),
            pl.BlockSpec((U_DIM, ufd), lambda i: (0, 0)),
            pl.BlockSpec((U_DIM, 1), lambda i: (0, 0)),
            pl.BlockSpec((NT, 3 * U_DIM), lambda i: (0, 0)),
            pl.BlockSpec((NT, 1), lambda i: (0, 0)),
        ],
        out_specs=pl.BlockSpec((NT, blk), lambda i: (0, i)),
        out_shape=jax.ShapeDtypeStruct((NT, B), jnp.float32),
    )(eu[0].reshape(128, -1), eu[1].reshape(128, -1), eu[2].reshape(128, -1),
      ei[0].reshape(128, -1), ei[1].reshape(128, -1), ei[2].reshape(128, -1),
      uf, W_ufT, b_uf, W_finalT, b_finalT)
    return outT.T


def kernel(user_id, user_features, item_id, user_table, item_table,
           W_uf, b_uf, W_final, b_final):
    uid = user_id.astype(jnp.int32)
    iid = item_id.astype(jnp.int32)
    pi = _tc_project(item_table.T, W_final[2 * U_DIM:, :].T)
    ei = _sc_gather(pi, iid)
    pu = _tc_project(user_table.T, W_final[0:U_DIM, :].T)
    eu = _sc_gather(pu, uid)
    return _tc_combine(eu, ei, user_features, W_uf.T,
                       b_uf.reshape(-1, 1), W_final.T,
                       b_final.reshape(-1, 1))
